# Initial kernel scaffold; baseline (speedup 1.0000x reference)
import jax
import jax.numpy as jnp
from jax.experimental import pallas as pl


def kernel(patches, patch_positions, Wqkv, Wproj, bproj):
    raise NotImplementedError("write your pallas kernel here")


# speedup vs baseline: 1.1874x; 1.1874x over previous
import functools

import jax
import jax.numpy as jnp
from jax import lax
from jax.experimental import pallas as pl

DIM = 128
NUM_HEADS = 8
HEAD_DIM = DIM // NUM_HEADS
SCALE = HEAD_DIM ** -0.5
PATCH = (2, 7, 7)
NUM_CONN = 4
LOCAL_RADIUS = 1
SPATIAL_SIGMA = 32.0
TEMPORAL_SIGMA = 2.0

N = 1024
V = 98
BLK_N = 128


def _stats_body(p_ref, x_ref, d_ref):
    p = p_ref[...]
    x_ref[...] = jnp.mean(p, axis=1)
    norms = jnp.sqrt(jnp.sum(p * p, axis=-1))
    d_ref[...] = jnp.mean(norms, axis=-1).reshape(1, 1, BLK_N)


def _attn_body(x_ref, dr_ref, dc_ref, tr_ref, hr_ref, wr_ref,
               tc_ref, hc_ref, wc_ref, wqkv_ref, wproj_ref, bproj_ref,
               out_ref):
    dr = dr_ref[...]
    dc = dc_ref[...]
    dmax = jnp.max(dc) + 1e-8
    dr = dr / dmax
    dc = dc / dmax

    td = jnp.abs(tc_ref[...] - tr_ref[...]) * float(PATCH[0])
    hd = jnp.abs(hc_ref[...] - hr_ref[...]) * float(PATCH[1])
    wd = jnp.abs(wc_ref[...] - wr_ref[...]) * float(PATCH[2])
    spatial = jnp.exp(-jnp.sqrt(hd * hd + wd * wd) / SPATIAL_SIGMA)
    temporal = jnp.exp(-td / TEMPORAL_SIGMA)
    scores = jnp.sqrt(dc * dr + 1e-8) + spatial * temporal

    colid = lax.broadcasted_iota(jnp.int32, (N, N), 1)
    rowid = lax.broadcasted_iota(jnp.int32, (N, N), 0)
    mask = jnp.abs(rowid - colid) <= LOCAL_RADIUS

    s = scores
    for _ in range(NUM_CONN):
        mx = jnp.max(s, axis=1, keepdims=True)
        jstar = jnp.min(jnp.where(s == mx, colid, N), axis=1, keepdims=True)
        hit = colid == jstar
        mask = mask | hit
        s = jnp.where(hit, -jnp.inf, s)

    x = x_ref[...]
    qkv = jnp.dot(x, wqkv_ref[...], preferred_element_type=jnp.float32)

    outs = []
    for h in range(NUM_HEADS):
        q = qkv[:, h * HEAD_DIM:(h + 1) * HEAD_DIM]
        k = qkv[:, DIM + h * HEAD_DIM: DIM + (h + 1) * HEAD_DIM]
        v = qkv[:, 2 * DIM + h * HEAD_DIM: 2 * DIM + (h + 1) * HEAD_DIM]
        logits = lax.dot_general(q, k, (((1,), (1,)), ((), ())),
                                 preferred_element_type=jnp.float32) * SCALE
        logits = jnp.where(mask, logits, -1e9)
        mx = jnp.max(logits, axis=1, keepdims=True)
        e = jnp.exp(logits - mx)
        a = e / jnp.sum(e, axis=1, keepdims=True)
        outs.append(jnp.dot(a, v, preferred_element_type=jnp.float32))
    o = jnp.concatenate(outs, axis=1)
    out_ref[...] = (jnp.dot(o, wproj_ref[...], preferred_element_type=jnp.float32)
                    + bproj_ref[...])


@functools.partial(jax.jit, static_argnames=("interpret",))
def kernel(patches, patch_positions, Wqkv, Wproj, bproj, interpret=False):
    B, n, v, c = patches.shape
    p3 = patches.reshape(n, v, c)

    x, dens = pl.pallas_call(
        _stats_body,
        grid=(n // BLK_N,),
        in_specs=[pl.BlockSpec((BLK_N, v, c), lambda i: (i, 0, 0))],
        out_specs=[
            pl.BlockSpec((BLK_N, c), lambda i: (i, 0)),
            pl.BlockSpec((1, 1, BLK_N), lambda i: (i, 0, 0)),
        ],
        out_shape=[
            jax.ShapeDtypeStruct((n, c), jnp.float32),
            jax.ShapeDtypeStruct((n // BLK_N, 1, BLK_N), jnp.float32),
        ],
        interpret=interpret,
    )(p3)

    dens = dens.reshape(n)
    pos = patch_positions.astype(jnp.float32)

    args = (
        x,
        dens.reshape(1, n), dens.reshape(n, 1),
        pos[:, 0].reshape(1, n), pos[:, 1].reshape(1, n), pos[:, 2].reshape(1, n),
        pos[:, 0].reshape(n, 1), pos[:, 1].reshape(n, 1), pos[:, 2].reshape(n, 1),
        Wqkv, Wproj, bproj.reshape(1, c),
    )
    out = pl.pallas_call(
        _attn_body,
        in_specs=[pl.BlockSpec(a.shape, lambda: (0,) * a.ndim) for a in args],
        out_specs=pl.BlockSpec((n, c), lambda: (0, 0)),
        out_shape=jax.ShapeDtypeStruct((n, c), jnp.float32),
        interpret=interpret,
    )(*args)
    return out.reshape(B, n, c)


# speedup vs baseline: 1.2553x; 1.0571x over previous
import functools

import jax
import jax.numpy as jnp
from jax import lax
from jax.experimental import pallas as pl

DIM = 128
NUM_HEADS = 8
HEAD_DIM = DIM // NUM_HEADS
SCALE = HEAD_DIM ** -0.5
PATCH = (2, 7, 7)
NUM_CONN = 4
LOCAL_RADIUS = 1
SPATIAL_SIGMA = 32.0
TEMPORAL_SIGMA = 2.0

N = 1024
V = 98
BLK_N = 128


def _stats_body(p_ref, x_ref, d_ref):
    p = p_ref[0]
    x_ref[...] = jnp.mean(p, axis=1)
    norms = jnp.sqrt(jnp.sum(p * p, axis=-1))
    d_ref[...] = jnp.mean(norms, axis=-1).reshape(1, 1, BLK_N)


def _attn_body(x_ref, dr_ref, dc_ref, tr_ref, hr_ref, wr_ref,
               tc_ref, hc_ref, wc_ref, wqkv_ref, wproj_ref, bproj_ref,
               out_ref):
    dr = dr_ref[...]
    dc = dc_ref[...]
    dmax = jnp.max(dc) + 1e-8
    dr = dr / dmax
    dc = dc / dmax

    td = jnp.abs(tc_ref[...] - tr_ref[...]) * float(PATCH[0])
    hd = jnp.abs(hc_ref[...] - hr_ref[...]) * float(PATCH[1])
    wd = jnp.abs(wc_ref[...] - wr_ref[...]) * float(PATCH[2])
    spatial = jnp.exp(-jnp.sqrt(hd * hd + wd * wd) / SPATIAL_SIGMA)
    temporal = jnp.exp(-td / TEMPORAL_SIGMA)
    scores = jnp.sqrt(dc * dr + 1e-8) + spatial * temporal

    colid = lax.broadcasted_iota(jnp.int32, (N, N), 1)
    rowid = lax.broadcasted_iota(jnp.int32, (N, N), 0)
    mask = jnp.abs(rowid - colid) <= LOCAL_RADIUS

    s = scores
    for _ in range(NUM_CONN):
        mx = jnp.max(s, axis=1, keepdims=True)
        jstar = jnp.min(jnp.where(s == mx, colid, N), axis=1, keepdims=True)
        hit = colid == jstar
        mask = mask | hit
        s = jnp.where(hit, -jnp.inf, s)

    madd = jnp.where(mask, 0.0, -1e9)
    x = x_ref[...]
    qkv = jnp.dot(x, wqkv_ref[...], preferred_element_type=jnp.float32)

    outs = []
    for h in range(NUM_HEADS):
        q = qkv[:, h * HEAD_DIM:(h + 1) * HEAD_DIM] * SCALE
        k = qkv[:, DIM + h * HEAD_DIM: DIM + (h + 1) * HEAD_DIM]
        v = qkv[:, 2 * DIM + h * HEAD_DIM: 2 * DIM + (h + 1) * HEAD_DIM]
        logits = lax.dot_general(q, k, (((1,), (1,)), ((), ())),
                                 preferred_element_type=jnp.float32) + madd
        mx = jnp.max(logits, axis=1, keepdims=True)
        e = jnp.exp(logits - mx)
        a = e * (1.0 / jnp.sum(e, axis=1, keepdims=True))
        outs.append(jnp.dot(a, v, preferred_element_type=jnp.float32))
    o = jnp.concatenate(outs, axis=1)
    out_ref[...] = (jnp.dot(o, wproj_ref[...], preferred_element_type=jnp.float32)
                    + bproj_ref[...])


@functools.partial(jax.jit, static_argnames=("interpret",))
def kernel(patches, patch_positions, Wqkv, Wproj, bproj, interpret=False):
    B, n, v, c = patches.shape

    x, dens = pl.pallas_call(
        _stats_body,
        grid=(n // BLK_N,),
        in_specs=[pl.BlockSpec((1, BLK_N, v, c), lambda i: (0, i, 0, 0))],
        out_specs=[
            pl.BlockSpec((BLK_N, c), lambda i: (i, 0)),
            pl.BlockSpec((1, 1, BLK_N), lambda i: (i, 0, 0)),
        ],
        out_shape=[
            jax.ShapeDtypeStruct((n, c), jnp.float32),
            jax.ShapeDtypeStruct((n // BLK_N, 1, BLK_N), jnp.float32),
        ],
        interpret=interpret,
    )(patches)

    dens = dens.reshape(n)
    pos = patch_positions.astype(jnp.float32)

    args = (
        x,
        dens.reshape(1, n), dens.reshape(n, 1),
        pos[:, 0].reshape(1, n), pos[:, 1].reshape(1, n), pos[:, 2].reshape(1, n),
        pos[:, 0].reshape(n, 1), pos[:, 1].reshape(n, 1), pos[:, 2].reshape(n, 1),
        Wqkv, Wproj, bproj.reshape(1, c),
    )
    out = pl.pallas_call(
        _attn_body,
        in_specs=[pl.BlockSpec(a.shape, lambda: (0,) * a.ndim) for a in args],
        out_specs=pl.BlockSpec((n, c), lambda: (0, 0)),
        out_shape=jax.ShapeDtypeStruct((n, c), jnp.float32),
        interpret=interpret,
    )(*args)
    return out.reshape(B, n, c)


# speedup vs baseline: 1.5147x; 1.2066x over previous
import functools

import jax
import jax.numpy as jnp
from jax import lax
from jax.experimental import pallas as pl
from jax.experimental.pallas import tpu as pltpu

DIM = 128
NUM_HEADS = 8
HEAD_DIM = DIM // NUM_HEADS
SCALE = HEAD_DIM ** -0.5
PATCH = (2, 7, 7)
NUM_CONN = 4
SPATIAL_SIGMA = 32.0
TEMPORAL_SIGMA = 2.0

N = 1024
V = 98
BLK_N = 128
GRID = N // BLK_N

NEG = -1e9


def _body(p_ref, tr_ref, hr_ref, wr_ref, tc_ref, hc_ref, wc_ref,
          wqkv_ref, wproj_ref, bproj_ref, out_ref,
          x_s, dc_s, dr_s, prox_s):
    i = pl.program_id(0)
    r0 = i * BLK_N

    p = p_ref[0]
    x_s[pl.ds(r0, BLK_N), :] = jnp.mean(p, axis=1)
    norms = jnp.sqrt(jnp.sum(p * p, axis=-1))
    dblk = jnp.mean(norms, axis=-1)
    dc_s[pl.ds(r0, BLK_N), :] = dblk.reshape(BLK_N, 1)
    dr_s[:, pl.ds(r0, BLK_N)] = dblk.reshape(1, BLK_N)

    tcb = tc_ref[pl.ds(r0, BLK_N), :]
    hcb = hc_ref[pl.ds(r0, BLK_N), :]
    wcb = wc_ref[pl.ds(r0, BLK_N), :]
    td = jnp.abs(tcb - tr_ref[...]) * float(PATCH[0])
    hd = jnp.abs(hcb - hr_ref[...]) * float(PATCH[1])
    wd = jnp.abs(wcb - wr_ref[...]) * float(PATCH[2])
    prox_s[pl.ds(r0, BLK_N), :] = (
        jnp.exp(-jnp.sqrt(hd * hd + wd * wd) / SPATIAL_SIGMA)
        * jnp.exp(-td / TEMPORAL_SIGMA))

    @pl.when(i == GRID - 1)
    def _phase2():
        dc = dc_s[...]
        dr = dr_s[...]
        dmax = jnp.max(dc) + 1e-8
        dc = dc / dmax
        dr = dr / dmax
        scores = jnp.sqrt(dc * dr + 1e-8) + prox_s[...]

        colid = lax.broadcasted_iota(jnp.int32, (N, N), 1)
        rowcol = lax.broadcasted_iota(jnp.int32, (N, 1), 0)

        x = x_s[...]
        qkv = jnp.dot(x, wqkv_ref[...], preferred_element_type=jnp.float32)
        qS = qkv[:, 0:DIM] * SCALE
        kall = qkv[:, DIM:2 * DIM]
        vall = qkv[:, 2 * DIM:3 * DIM]

        hid = lax.broadcasted_iota(jnp.int32, (DIM, NUM_HEADS), 0) // HEAD_DIM
        hcols = lax.broadcasted_iota(jnp.int32, (DIM, NUM_HEADS), 1)
        segR = jnp.where(hid == hcols, 1.0, 0.0)
        eid = lax.broadcasted_iota(jnp.int32, (NUM_HEADS, DIM), 1) // HEAD_DIM
        erow = lax.broadcasted_iota(jnp.int32, (NUM_HEADS, DIM), 0)
        segX = jnp.where(eid == erow, 1.0, 0.0)

        zrow = jnp.zeros((1, DIM), jnp.float32)
        k_m1 = jnp.concatenate([zrow, kall[:-1, :]], axis=0)
        k_p1 = jnp.concatenate([kall[1:, :], zrow], axis=0)
        v_m1 = jnp.concatenate([zrow, vall[:-1, :]], axis=0)
        v_p1 = jnp.concatenate([vall[1:, :], zrow], axis=0)

        def seg_logit(kmat):
            return jnp.dot(qS * kmat, segR, preferred_element_type=jnp.float32)

        l_m1 = seg_logit(k_m1)
        l_00 = seg_logit(kall)
        l_p1 = seg_logit(k_p1)
        valid_m1 = rowcol >= 1
        valid_p1 = rowcol <= N - 2
        l_m1 = jnp.where(valid_m1, l_m1, NEG)
        l_p1 = jnp.where(valid_p1, l_p1, NEG)

        s = scores
        l_c, v_c, dup_c = [], [], []
        for _ in range(NUM_CONN):
            mx = jnp.max(s, axis=1, keepdims=True)
            jstar = jnp.min(jnp.where(s == mx, colid, N), axis=1,
                            keepdims=True)
            hit = colid == jstar
            s = jnp.where(hit, -jnp.inf, s)
            hit_f = jnp.where(hit, 1.0, 0.0)
            kg = jnp.dot(hit_f, kall, preferred_element_type=jnp.float32)
            vg = jnp.dot(hit_f, vall, preferred_element_type=jnp.float32)
            l_c.append(seg_logit(kg))
            v_c.append(vg)
            dup_c.append(jnp.abs(jstar - rowcol) <= 1)

        mx = jnp.maximum(jnp.maximum(l_m1, l_00), l_p1)
        for lc in l_c:
            mx = jnp.maximum(mx, lc)
        e_m1 = jnp.where(valid_m1, jnp.exp(l_m1 - mx), 0.0)
        e_00 = jnp.exp(l_00 - mx)
        e_p1 = jnp.where(valid_p1, jnp.exp(l_p1 - mx), 0.0)
        denom = e_m1 + e_00 + e_p1
        e_cs = []
        for lc, dup in zip(l_c, dup_c):
            ec = jnp.where(dup, 0.0, jnp.exp(lc - mx))
            e_cs.append(ec)
            denom = denom + ec
        rinv = 1.0 / denom

        def expand(w):
            return jnp.dot(w, segX, preferred_element_type=jnp.float32)

        o = (expand(e_m1 * rinv) * v_m1 + expand(e_00 * rinv) * vall
             + expand(e_p1 * rinv) * v_p1)
        for ec, vg in zip(e_cs, v_c):
            o = o + expand(ec * rinv) * vg

        out_ref[...] = (jnp.dot(o, wproj_ref[...],
                                preferred_element_type=jnp.float32)
                        + bproj_ref[...])


@functools.partial(jax.jit, static_argnames=("interpret",))
def kernel(patches, patch_positions, Wqkv, Wproj, bproj, interpret=False):
    B, n, v, c = patches.shape
    pos = patch_positions.astype(jnp.float32)

    args = (
        patches,
        pos[:, 0].reshape(1, n), pos[:, 1].reshape(1, n), pos[:, 2].reshape(1, n),
        pos[:, 0].reshape(n, 1), pos[:, 1].reshape(n, 1), pos[:, 2].reshape(n, 1),
        Wqkv, Wproj, bproj.reshape(1, c),
    )
    in_specs = [pl.BlockSpec((1, BLK_N, v, c), lambda i: (0, i, 0, 0))]
    in_specs += [pl.BlockSpec(a.shape, lambda i: (0,) * a.ndim)
                 for a in args[1:]]
    out = pl.pallas_call(
        _body,
        grid=(GRID,),
        in_specs=in_specs,
        out_specs=pl.BlockSpec((n, c), lambda i: (0, 0)),
        out_shape=jax.ShapeDtypeStruct((n, c), jnp.float32),
        scratch_shapes=[
            pltpu.VMEM((n, c), jnp.float32),
            pltpu.VMEM((n, 1), jnp.float32),
            pltpu.VMEM((1, n), jnp.float32),
            pltpu.VMEM((n, n), jnp.float32),
        ],
        interpret=interpret,
    )(*args)
    return out.reshape(B, n, c)
